# baseline (device time: 598513 ns/iter reference)
import jax
import jax.numpy as jnp
from jax import lax
from jax.experimental import pallas as pl
from jax.experimental.pallas import tpu as pltpu

E = 8
E_LOC = 4
C = 544
D = 2048
F = 4096
FT = 256
NFT = F // FT
T = 4096
S = E * C
HALF = E_LOC * C
R_IN = 256
R_OUT = 256

NS = 8


def _le_of(s):
    return jnp.where(s < 3, s, jnp.where(s < 7, s - 3, 3))


def _cast_body(x_ref, o_ref):
    o_ref[...] = x_ref[...].astype(jnp.bfloat16)


def _cast_bf16(w, rows):
    n, a, b = w.shape
    return pl.pallas_call(
        _cast_body,
        grid=(n, a // rows),
        in_specs=[pl.BlockSpec((1, rows, b), lambda i, j: (i, j, 0))],
        out_specs=pl.BlockSpec((1, rows, b), lambda i, j: (i, j, 0)),
        out_shape=jax.ShapeDtypeStruct((n, a, b), jnp.bfloat16),
    )(w)


def _permute_in_body(slot_ref, x_ref, o_ref):
    i = pl.program_id(0)
    rid = i * R_IN + lax.broadcasted_iota(jnp.int32, (R_IN, T), 0)
    mask = (rid == slot_ref[...]).astype(jnp.bfloat16)
    o_ref[...] = jnp.dot(
        mask, x_ref[...], preferred_element_type=jnp.float32
    ).astype(jnp.bfloat16)


def _permute_in(xb, slot_row):
    return pl.pallas_call(
        _permute_in_body,
        grid=(S // R_IN,),
        in_specs=[
            pl.BlockSpec((1, T), lambda i: (0, 0)),
            pl.BlockSpec((T, D), lambda i: (0, 0)),
        ],
        out_specs=pl.BlockSpec((R_IN, D), lambda i: (i, 0)),
        out_shape=jax.ShapeDtypeStruct((S, D), jnp.bfloat16),
    )(slot_row, xb)


def _permute_out_body(slot_ref, loc_ref, rem_ref, o_ref):
    sl = slot_ref[...]
    cols = lax.broadcasted_iota(jnp.int32, (R_OUT, HALF), 1)
    m_l = (sl == cols).astype(jnp.bfloat16)
    m_r = (sl == cols + HALF).astype(jnp.bfloat16)
    o_ref[...] = jnp.dot(
        m_l, loc_ref[...], preferred_element_type=jnp.float32
    ) + jnp.dot(m_r, rem_ref[...], preferred_element_type=jnp.float32)


def _permute_out(out_local, out_remote, slot_col):
    return pl.pallas_call(
        _permute_out_body,
        grid=(T // R_OUT,),
        in_specs=[
            pl.BlockSpec((R_OUT, 1), lambda i: (i, 0)),
            pl.BlockSpec((HALF, D), lambda i: (0, 0)),
            pl.BlockSpec((HALF, D), lambda i: (0, 0)),
        ],
        out_specs=pl.BlockSpec((R_OUT, D), lambda i: (i, 0)),
        out_shape=jax.ShapeDtypeStruct((T, D), jnp.float32),
    )(slot_col, out_local.reshape(HALF, D), out_remote.reshape(HALF, D))


def _body(buf_local, buf_foreign, w1_ref, w2_ref, out_local, out_remote,
          recv_ref, acc, tokens, dsend, drecv, rsend, rrecv, csem, lsem):
    s = pl.program_id(0)
    ft = pl.program_id(1)
    le = _le_of(s)
    p = (s >= 3) & (s < 7)
    my_x = lax.axis_index("x")
    my_y = lax.axis_index("y")
    px = 1 - my_x

    def disp_rdma(i):
        return pltpu.make_async_remote_copy(
            src_ref=buf_foreign.at[i],
            dst_ref=recv_ref.at[i],
            send_sem=dsend.at[i],
            recv_sem=drecv.at[i],
            device_id=(px, my_y),
            device_id_type=pl.DeviceIdType.MESH,
        )

    def res_rdma(i):
        return pltpu.make_async_remote_copy(
            src_ref=recv_ref.at[i],
            dst_ref=out_remote.at[i],
            send_sem=rsend.at[i],
            recv_sem=rrecv.at[i],
            device_id=(px, my_y),
            device_id_type=pl.DeviceIdType.MESH,
        )

    @pl.when((s == 0) & (ft == 0))
    def _():
        barrier = pltpu.get_barrier_semaphore()
        pl.semaphore_signal(
            barrier, inc=1,
            device_id=(px, my_y), device_id_type=pl.DeviceIdType.MESH,
        )
        pl.semaphore_wait(barrier, 1)
        for i in range(E_LOC):
            disp_rdma(i).start()

    for i in range(E_LOC):
        @pl.when((s == 3 + i) & (ft == 0))
        def _(i=i):
            disp_rdma(i).wait()

    @pl.when((ft == 0) & jnp.logical_not(p))
    def _():
        cp = pltpu.make_async_copy(buf_local.at[le], tokens, lsem)
        cp.start()
        cp.wait()

    @pl.when((ft == 0) & p)
    def _():
        tokens[...] = recv_ref[le]

    w1t = w1_ref[0].astype(jnp.bfloat16)
    w2t = w2_ref[0].astype(jnp.bfloat16)
    h = jnp.maximum(
        jnp.dot(tokens[...], w1t, preferred_element_type=jnp.float32), 0.0
    )
    part = jnp.dot(
        h.astype(jnp.bfloat16), w2t, preferred_element_type=jnp.float32
    )

    @pl.when(ft == 0)
    def _():
        acc[...] = part

    @pl.when(ft != 0)
    def _():
        acc[...] = acc[...] + part

    @pl.when((ft == NFT - 1) & jnp.logical_not(p))
    def _():
        tokens[...] = acc[...].astype(jnp.bfloat16)
        cp = pltpu.make_async_copy(tokens, out_local.at[le], csem)
        cp.start()
        cp.wait()

    for i in range(E_LOC):
        @pl.when((s == 3 + i) & (ft == NFT - 1))
        def _(i=i):
            recv_ref[i] = acc[...].astype(jnp.bfloat16)
            res_rdma(i).start()

    @pl.when((s == NS - 1) & (ft == NFT - 1))
    def _():
        for i in range(E_LOC):
            res_rdma(i).wait()


def kernel(x, assign, W1, W2):
    assert x.shape == (T, D) and W1.shape == (E_LOC, D, F)

    my_x = lax.axis_index("x")
    assign = assign.astype(jnp.int32)
    xb = _cast_bf16(x.reshape(1, T, D), 512).reshape(T, D)

    group = jnp.where(
        assign // E_LOC == my_x, assign % E_LOC, E_LOC + assign % E_LOC
    )
    onehot = (group[:, None] == jnp.arange(E, dtype=jnp.int32)[None, :])
    csum = jnp.cumsum(onehot.astype(jnp.int32), axis=0) - 1
    rank = jnp.sum(jnp.where(onehot, csum, 0), axis=1)
    slot = jnp.where(rank < C, group * C + rank, S)

    buf = _permute_in(xb, slot.reshape(1, T))
    buf_local = buf[:HALF].reshape(E_LOC, C, D)
    buf_foreign = buf[HALF:].reshape(E_LOC, C, D)

    grid = (NS, NFT)
    out_local, out_remote = pl.pallas_call(
        _body,
        grid=grid,
        in_specs=[
            pl.BlockSpec(memory_space=pltpu.MemorySpace.HBM),
            pl.BlockSpec(memory_space=pltpu.MemorySpace.HBM),
            pl.BlockSpec((1, D, FT), lambda s, ft: (_le_of(s), 0, ft)),
            pl.BlockSpec((1, FT, D), lambda s, ft: (_le_of(s), ft, 0)),
        ],
        out_specs=[
            pl.BlockSpec(memory_space=pltpu.MemorySpace.HBM),
            pl.BlockSpec(memory_space=pltpu.MemorySpace.HBM),
        ],
        out_shape=[
            jax.ShapeDtypeStruct((E_LOC, C, D), jnp.bfloat16),
            jax.ShapeDtypeStruct((E_LOC, C, D), jnp.bfloat16),
        ],
        scratch_shapes=[
            pltpu.VMEM((E_LOC, C, D), jnp.bfloat16),
            pltpu.VMEM((C, D), jnp.float32),
            pltpu.VMEM((C, D), jnp.bfloat16),
            pltpu.SemaphoreType.DMA((E_LOC,)),
            pltpu.SemaphoreType.DMA((E_LOC,)),
            pltpu.SemaphoreType.DMA((E_LOC,)),
            pltpu.SemaphoreType.DMA((E_LOC,)),
            pltpu.SemaphoreType.DMA,
            pltpu.SemaphoreType.DMA,
        ],
        compiler_params=pltpu.CompilerParams(
            collective_id=0,
            dimension_semantics=("arbitrary", "arbitrary"),
        ),
    )(buf_local, buf_foreign, W1, W2)

    return _permute_out(out_local, out_remote, slot.reshape(T, 1))


# device time: 535152 ns/iter; 1.1184x vs baseline; 1.1184x over previous
import jax
import jax.numpy as jnp
from jax import lax
from jax.experimental import pallas as pl
from jax.experimental.pallas import tpu as pltpu

E = 8
E_LOC = 4
C = 544
D = 2048
F = 4096
FT = 512
NFT = F // FT
T = 4096
S = E * C
HALF = E_LOC * C
R_IN = 256
R_OUT = 256

NS = 8


def _le_of(s):
    return jnp.where(s < 3, s, jnp.where(s < 7, s - 3, 3))


def _cast_body(x_ref, o_ref):
    o_ref[...] = x_ref[...].astype(jnp.bfloat16)


def _cast_bf16(w, rows):
    n, a, b = w.shape
    return pl.pallas_call(
        _cast_body,
        grid=(n, a // rows),
        in_specs=[pl.BlockSpec((1, rows, b), lambda i, j: (i, j, 0))],
        out_specs=pl.BlockSpec((1, rows, b), lambda i, j: (i, j, 0)),
        out_shape=jax.ShapeDtypeStruct((n, a, b), jnp.bfloat16),
    )(w)


def _permute_in_body(slot_ref, x_ref, o_ref):
    i = pl.program_id(0)
    rid = i * R_IN + lax.broadcasted_iota(jnp.int32, (R_IN, T), 0)
    mask = (rid == slot_ref[...]).astype(jnp.bfloat16)
    o_ref[...] = jnp.dot(
        mask, x_ref[...], preferred_element_type=jnp.float32
    ).astype(jnp.bfloat16)


def _permute_in(xb, slot_row):
    return pl.pallas_call(
        _permute_in_body,
        grid=(S // R_IN,),
        in_specs=[
            pl.BlockSpec((1, T), lambda i: (0, 0)),
            pl.BlockSpec((T, D), lambda i: (0, 0)),
        ],
        out_specs=pl.BlockSpec((R_IN, D), lambda i: (i, 0)),
        out_shape=jax.ShapeDtypeStruct((S, D), jnp.bfloat16),
    )(slot_row, xb)


def _permute_out_body(slot_ref, loc_ref, rem_ref, o_ref):
    sl = slot_ref[...]
    cols = lax.broadcasted_iota(jnp.int32, (R_OUT, HALF), 1)
    m_l = (sl == cols).astype(jnp.bfloat16)
    m_r = (sl == cols + HALF).astype(jnp.bfloat16)
    o_ref[...] = jnp.dot(
        m_l, loc_ref[...], preferred_element_type=jnp.float32
    ) + jnp.dot(m_r, rem_ref[...], preferred_element_type=jnp.float32)


def _permute_out(out_local, out_remote, slot_col):
    return pl.pallas_call(
        _permute_out_body,
        grid=(T // R_OUT,),
        in_specs=[
            pl.BlockSpec((R_OUT, 1), lambda i: (i, 0)),
            pl.BlockSpec((HALF, D), lambda i: (0, 0)),
            pl.BlockSpec((HALF, D), lambda i: (0, 0)),
        ],
        out_specs=pl.BlockSpec((R_OUT, D), lambda i: (i, 0)),
        out_shape=jax.ShapeDtypeStruct((T, D), jnp.float32),
    )(slot_col, out_local.reshape(HALF, D), out_remote.reshape(HALF, D))


def _body(buf_local, buf_foreign, w1_ref, w2_ref, out_local, out_remote,
          recv_ref, acc, tokens, dsend, drecv, rsend, rrecv, csem, lsem):
    s = pl.program_id(0)
    ft = pl.program_id(1)
    le = _le_of(s)
    p = (s >= 3) & (s < 7)
    my_x = lax.axis_index("x")
    my_y = lax.axis_index("y")
    px = 1 - my_x

    def disp_rdma(i):
        return pltpu.make_async_remote_copy(
            src_ref=buf_foreign.at[i],
            dst_ref=recv_ref.at[i],
            send_sem=dsend.at[i],
            recv_sem=drecv.at[i],
            device_id=(px, my_y),
            device_id_type=pl.DeviceIdType.MESH,
        )

    def res_rdma(i):
        return pltpu.make_async_remote_copy(
            src_ref=recv_ref.at[i],
            dst_ref=out_remote.at[i],
            send_sem=rsend.at[i],
            recv_sem=rrecv.at[i],
            device_id=(px, my_y),
            device_id_type=pl.DeviceIdType.MESH,
        )

    @pl.when((s == 0) & (ft == 0))
    def _():
        barrier = pltpu.get_barrier_semaphore()
        pl.semaphore_signal(
            barrier, inc=1,
            device_id=(px, my_y), device_id_type=pl.DeviceIdType.MESH,
        )
        pl.semaphore_wait(barrier, 1)
        for i in range(E_LOC):
            disp_rdma(i).start()

    for i in range(E_LOC):
        @pl.when((s == 3 + i) & (ft == 0))
        def _(i=i):
            disp_rdma(i).wait()

    @pl.when((ft == 0) & jnp.logical_not(p))
    def _():
        cp = pltpu.make_async_copy(buf_local.at[le], tokens, lsem)
        cp.start()
        cp.wait()

    @pl.when((ft == 0) & p)
    def _():
        tokens[...] = recv_ref[le]

    w1t = w1_ref[0]
    w2t = w2_ref[0]
    h = jnp.maximum(
        jnp.dot(tokens[...], w1t, preferred_element_type=jnp.float32), 0.0
    )
    part = jnp.dot(
        h.astype(jnp.bfloat16), w2t, preferred_element_type=jnp.float32
    )

    @pl.when(ft == 0)
    def _():
        acc[...] = part

    @pl.when(ft != 0)
    def _():
        acc[...] = acc[...] + part

    @pl.when((ft == NFT - 1) & jnp.logical_not(p))
    def _():
        tokens[...] = acc[...].astype(jnp.bfloat16)
        cp = pltpu.make_async_copy(tokens, out_local.at[le], csem)
        cp.start()
        cp.wait()

    for i in range(E_LOC):
        @pl.when((s == 3 + i) & (ft == NFT - 1))
        def _(i=i):
            recv_ref[i] = acc[...].astype(jnp.bfloat16)
            res_rdma(i).start()

    @pl.when((s == NS - 1) & (ft == NFT - 1))
    def _():
        for i in range(E_LOC):
            res_rdma(i).wait()


def kernel(x, assign, W1, W2):
    assert x.shape == (T, D) and W1.shape == (E_LOC, D, F)

    my_x = lax.axis_index("x")
    assign = assign.astype(jnp.int32)
    xb = _cast_bf16(x.reshape(1, T, D), 512).reshape(T, D)

    group = jnp.where(
        assign // E_LOC == my_x, assign % E_LOC, E_LOC + assign % E_LOC
    )
    onehot = (group[:, None] == jnp.arange(E, dtype=jnp.int32)[None, :])
    csum = jnp.cumsum(onehot.astype(jnp.int32), axis=0) - 1
    rank = jnp.sum(jnp.where(onehot, csum, 0), axis=1)
    slot = jnp.where(rank < C, group * C + rank, S)

    buf = _permute_in(xb, slot.reshape(1, T))
    buf_local = buf[:HALF].reshape(E_LOC, C, D)
    buf_foreign = buf[HALF:].reshape(E_LOC, C, D)

    grid = (NS, NFT)
    out_local, out_remote = pl.pallas_call(
        _body,
        grid=grid,
        in_specs=[
            pl.BlockSpec(memory_space=pltpu.MemorySpace.HBM),
            pl.BlockSpec(memory_space=pltpu.MemorySpace.HBM),
            pl.BlockSpec((1, D, FT), lambda s, ft: (_le_of(s), 0, ft)),
            pl.BlockSpec((1, FT, D), lambda s, ft: (_le_of(s), ft, 0)),
        ],
        out_specs=[
            pl.BlockSpec(memory_space=pltpu.MemorySpace.HBM),
            pl.BlockSpec(memory_space=pltpu.MemorySpace.HBM),
        ],
        out_shape=[
            jax.ShapeDtypeStruct((E_LOC, C, D), jnp.bfloat16),
            jax.ShapeDtypeStruct((E_LOC, C, D), jnp.bfloat16),
        ],
        scratch_shapes=[
            pltpu.VMEM((E_LOC, C, D), jnp.bfloat16),
            pltpu.VMEM((C, D), jnp.float32),
            pltpu.VMEM((C, D), jnp.bfloat16),
            pltpu.SemaphoreType.DMA((E_LOC,)),
            pltpu.SemaphoreType.DMA((E_LOC,)),
            pltpu.SemaphoreType.DMA((E_LOC,)),
            pltpu.SemaphoreType.DMA((E_LOC,)),
            pltpu.SemaphoreType.DMA,
            pltpu.SemaphoreType.DMA,
        ],
        compiler_params=pltpu.CompilerParams(
            collective_id=0,
            dimension_semantics=("arbitrary", "arbitrary"),
        ),
    )(buf_local, buf_foreign, _cast_bf16(W1, 256), _cast_bf16(W2, 512))

    return _permute_out(out_local, out_remote, slot.reshape(T, 1))
